# baseline (device time: 94004 ns/iter reference)
import jax
import jax.numpy as jnp
from jax import lax
from jax.experimental import pallas as pl
from jax.experimental.pallas import tpu as pltpu

N_DEV = 4
SQ = 512
D = 1024
H = 8
DH = 128
SKV = 2048
SCALE = 0.08838834764831843
NCH = 4
CH = SQ // NCH
BF = jnp.bfloat16
F32 = jnp.float32


def kernel(x, Wq, Wo, K_ext, V_ext):
    def body(x_ref, wq_ref, wo_ref, k_hbm, v_hbm, out_ref,
             xsend, xfull, partial, rsrecv, wqb, wob, qbuf, attn,
             kbuf, vbuf, kall, vall,
             ag_ssem, ag_rsem, rs_ssem, rs_rsem, rs2_ssem, rs2_rsem,
             k_sems, v_sems):
        my_i = lax.axis_index("i")
        left = lax.rem(my_i - 1 + N_DEV, N_DEV)
        right = lax.rem(my_i + 1, N_DEV)
        h0 = my_i * H

        barrier = pltpu.get_barrier_semaphore()
        for nbr in (left, right):
            pl.semaphore_signal(barrier, inc=1, device_id=(nbr,),
                                device_id_type=pl.DeviceIdType.MESH)
        xsend[:, :] = x_ref[:, :].astype(BF)
        pl.semaphore_wait(barrier, 2)

        def ag_rdma(h):
            return pltpu.make_async_remote_copy(
                src_ref=xsend if h == 0 else xfull.at[h - 1],
                dst_ref=xfull.at[h],
                send_sem=ag_ssem.at[h],
                recv_sem=ag_rsem.at[h],
                device_id=(right,),
                device_id_type=pl.DeviceIdType.MESH,
            )

        def rs_rdma(t):
            return pltpu.make_async_remote_copy(
                src_ref=partial.at[t],
                dst_ref=rsrecv.at[t],
                send_sem=rs_ssem.at[t],
                recv_sem=rs_rsem.at[t],
                device_id=(right,),
                device_id_type=pl.DeviceIdType.MESH,
            )

        def kv_copy(h, slot):
            return (
                pltpu.make_async_copy(
                    k_hbm.at[:, h0 + h, :], kbuf.at[slot], k_sems.at[slot]),
                pltpu.make_async_copy(
                    v_hbm.at[:, h0 + h, :], vbuf.at[slot], v_sems.at[slot]),
            )

        def attn_core(xsrc, stream, heads=range(H)):
            if stream:
                for c in kv_copy(0, 0):
                    c.start()
            if 0 in heads:
                qbuf[:, :] = jnp.dot(xsrc[:, :], wqb[:, :],
                                     preferred_element_type=F32).astype(BF)
            for h in heads:
                if stream:
                    cur = h % 2
                    if h + 1 < H:
                        for c in kv_copy(h + 1, (h + 1) % 2):
                            c.start()
                    for c in kv_copy(h, cur):
                        c.wait()
                    kall[h, :, :] = kbuf[cur].astype(BF)
                    vall[h, :, :] = vbuf[cur].astype(BF)
                qh = qbuf[:, h * DH:(h + 1) * DH]
                s = lax.dot_general(
                    qh, kall[h], (((1,), (1,)), ((), ())),
                    preferred_element_type=F32)
                p = jnp.exp(s).astype(BF)
                l = jnp.sum(p, axis=1, keepdims=True, dtype=F32)
                oh = jnp.dot(p, vall[h], preferred_element_type=F32)
                attn[:, h * DH:(h + 1) * DH] = (oh / l).astype(BF)

        def block_attn(xsrc, dst_ref, dst_f32, stream=False):
            attn_core(xsrc, stream)
            o = jnp.dot(attn[:, :], wob[:, :], preferred_element_type=F32)
            dst_ref[:, :] = o if dst_f32 else o.astype(BF)

        ag = [ag_rdma(h) for h in range(N_DEV - 1)]
        rs = [rs_rdma(t) for t in range(N_DEV - 1)]

        ag[0].start()
        wqb[:, :] = (wq_ref[:, :] * SCALE).astype(BF)
        wob[:, :] = wo_ref[:, :].astype(BF)
        block_attn(xsend, out_ref, True, stream=True)
        ag[0].wait_recv()
        ag[1].start()
        block_attn(xfull.at[0], partial.at[0], False)
        ag[1].wait_recv()
        ag[2].start()
        rs[0].start()
        block_attn(xfull.at[1], partial.at[1], False)
        last = N_DEV - 2
        ag[last].wait_recv()
        attn_core(xfull.at[last], False, heads=range(0, H // 2))
        rs[0].wait_recv()
        partial[1, :, :] = (
            partial[1].astype(F32) + rsrecv[0].astype(F32)
        ).astype(BF)
        rs[1].start()
        attn_core(xfull.at[last], False, heads=range(H // 2, H))
        rs[last - 1].wait_recv()
        rs2 = [
            pltpu.make_async_remote_copy(
                src_ref=partial.at[last, pl.ds(c * CH, CH)],
                dst_ref=rsrecv.at[last, pl.ds(c * CH, CH)],
                send_sem=rs2_ssem.at[c],
                recv_sem=rs2_rsem.at[c],
                device_id=(right,),
                device_id_type=pl.DeviceIdType.MESH,
            )
            for c in range(NCH)
        ]
        for c in range(NCH):
            r0, r1 = c * CH, (c + 1) * CH
            o = jnp.dot(attn[r0:r1, :], wob[:, :], preferred_element_type=F32)
            partial[last, r0:r1, :] = (
                o + rsrecv[last - 1, r0:r1, :].astype(F32)
            ).astype(BF)
            rs2[c].start()
        for c in range(NCH):
            r0, r1 = c * CH, (c + 1) * CH
            rs2[c].wait_recv()
            out_ref[r0:r1, :] = (
                out_ref[r0:r1, :] + rsrecv[last, r0:r1, :].astype(F32)
            )

        for r in ag + rs[:last] + rs2:
            r.wait_send()

    out = pl.pallas_call(
        body,
        out_shape=jax.ShapeDtypeStruct((SQ, D), F32),
        in_specs=[
            pl.BlockSpec(memory_space=pltpu.VMEM),
            pl.BlockSpec(memory_space=pltpu.VMEM),
            pl.BlockSpec(memory_space=pltpu.VMEM),
            pl.BlockSpec(memory_space=pltpu.MemorySpace.HBM),
            pl.BlockSpec(memory_space=pltpu.MemorySpace.HBM),
        ],
        out_specs=pl.BlockSpec(memory_space=pltpu.VMEM),
        scratch_shapes=[
            pltpu.VMEM((SQ, D), BF),
            pltpu.VMEM((N_DEV - 1, SQ, D), BF),
            pltpu.VMEM((N_DEV - 1, SQ, D), BF),
            pltpu.VMEM((N_DEV - 1, SQ, D), BF),
            pltpu.VMEM((D, D), BF),
            pltpu.VMEM((D, D), BF),
            pltpu.VMEM((SQ, D), BF),
            pltpu.VMEM((SQ, D), BF),
            pltpu.VMEM((2, SKV, DH), F32),
            pltpu.VMEM((2, SKV, DH), F32),
            pltpu.VMEM((H, SKV, DH), BF),
            pltpu.VMEM((H, SKV, DH), BF),
            pltpu.SemaphoreType.DMA((N_DEV - 1,)),
            pltpu.SemaphoreType.DMA((N_DEV - 1,)),
            pltpu.SemaphoreType.DMA((N_DEV - 1,)),
            pltpu.SemaphoreType.DMA((N_DEV - 1,)),
            pltpu.SemaphoreType.DMA((NCH,)),
            pltpu.SemaphoreType.DMA((NCH,)),
            pltpu.SemaphoreType.DMA((2,)),
            pltpu.SemaphoreType.DMA((2,)),
        ],
        compiler_params=pltpu.CompilerParams(
            collective_id=0, vmem_limit_bytes=63 * 1024 * 1024),
    )(x[0], Wq, Wo, K_ext[0], V_ext[0])
    return out[None]


# device time: 91819 ns/iter; 1.0238x vs baseline; 1.0238x over previous
import jax
import jax.numpy as jnp
from jax import lax
from jax.experimental import pallas as pl
from jax.experimental.pallas import tpu as pltpu

N_DEV = 4
SQ = 512
D = 1024
H = 8
DH = 128
SKV = 2048
SCALE = 0.08838834764831843
NCH = 4
CH = SQ // NCH
BF = jnp.bfloat16
F32 = jnp.float32


def kernel(x, Wq, Wo, K_ext, V_ext):
    def body(x_ref, wq_ref, wo_ref, k_hbm, v_hbm, out_ref,
             xsend, xfull, partial, rsrecv, wqb, wob, qbuf, attn,
             kbuf, vbuf, kall, vall,
             ag_ssem, ag_rsem, rs_ssem, rs_rsem, rs2_ssem, rs2_rsem,
             k_sems, v_sems):
        my_i = lax.axis_index("i")
        left = lax.rem(my_i - 1 + N_DEV, N_DEV)
        right = lax.rem(my_i + 1, N_DEV)
        h0 = my_i * H

        barrier = pltpu.get_barrier_semaphore()
        for nbr in (left, right):
            pl.semaphore_signal(barrier, inc=1, device_id=(nbr,),
                                device_id_type=pl.DeviceIdType.MESH)
        xsend[:, :] = x_ref[:, :].astype(BF)
        pl.semaphore_wait(barrier, 2)

        def ag_rdma(h):
            return pltpu.make_async_remote_copy(
                src_ref=xsend if h == 0 else xfull.at[h - 1],
                dst_ref=xfull.at[h],
                send_sem=ag_ssem.at[h],
                recv_sem=ag_rsem.at[h],
                device_id=(right,),
                device_id_type=pl.DeviceIdType.MESH,
            )

        def rs_rdma(t):
            return pltpu.make_async_remote_copy(
                src_ref=partial.at[t],
                dst_ref=rsrecv.at[t],
                send_sem=rs_ssem.at[t],
                recv_sem=rs_rsem.at[t],
                device_id=(right,),
                device_id_type=pl.DeviceIdType.MESH,
            )

        def kv_copy(h, slot):
            return (
                pltpu.make_async_copy(
                    k_hbm.at[:, h0 + h, :], kbuf.at[slot], k_sems.at[slot]),
                pltpu.make_async_copy(
                    v_hbm.at[:, h0 + h, :], vbuf.at[slot], v_sems.at[slot]),
            )

        def attn_core(xsrc, stream):
            if stream:
                for c in kv_copy(0, 0):
                    c.start()
            qbuf[:, :] = jnp.dot(xsrc[:, :], wqb[:, :],
                                 preferred_element_type=F32).astype(BF)
            for h in range(H):
                if stream:
                    cur = h % 2
                    if h + 1 < H:
                        for c in kv_copy(h + 1, (h + 1) % 2):
                            c.start()
                    for c in kv_copy(h, cur):
                        c.wait()
                    kall[h, :, :] = kbuf[cur].astype(BF)
                    vall[h, :, :] = vbuf[cur].astype(BF)
                qh = qbuf[:, h * DH:(h + 1) * DH]
                s = lax.dot_general(
                    qh, kall[h], (((1,), (1,)), ((), ())),
                    preferred_element_type=F32)
                p = jnp.exp(s).astype(BF)
                l = jnp.sum(p, axis=1, keepdims=True, dtype=F32)
                oh = jnp.dot(p, vall[h], preferred_element_type=F32)
                attn[:, h * DH:(h + 1) * DH] = (oh / l).astype(BF)

        def block_attn(xsrc, dst_ref, dst_f32, stream=False):
            attn_core(xsrc, stream)
            o = jnp.dot(attn[:, :], wob[:, :], preferred_element_type=F32)
            dst_ref[:, :] = o if dst_f32 else o.astype(BF)

        ag = [ag_rdma(h) for h in range(N_DEV - 1)]
        rs = [rs_rdma(t) for t in range(N_DEV - 1)]

        ag[0].start()
        wqb[:, :] = (wq_ref[:, :] * SCALE).astype(BF)
        wob[:, :] = wo_ref[:, :].astype(BF)
        block_attn(xsend, out_ref, True, stream=True)
        for j in range(N_DEV - 2):
            ag[j].wait_recv()
            ag[j + 1].start()
            block_attn(xfull.at[j], partial.at[j], False)
            if j > 0:
                rs[j - 1].wait_recv()
                partial[j, :, :] = (
                    partial[j].astype(F32) + rsrecv[j - 1].astype(F32)
                ).astype(BF)
            rs[j].start()

        last = N_DEV - 2
        ag[last].wait_recv()
        attn_core(xfull.at[last], False)
        rs[last - 1].wait_recv()
        rs2 = [
            pltpu.make_async_remote_copy(
                src_ref=partial.at[last, pl.ds(c * CH, CH)],
                dst_ref=rsrecv.at[last, pl.ds(c * CH, CH)],
                send_sem=rs2_ssem.at[c],
                recv_sem=rs2_rsem.at[c],
                device_id=(right,),
                device_id_type=pl.DeviceIdType.MESH,
            )
            for c in range(NCH)
        ]
        for c in range(NCH):
            r0, r1 = c * CH, (c + 1) * CH
            o = jnp.dot(attn[r0:r1, :], wob[:, :], preferred_element_type=F32)
            partial[last, r0:r1, :] = (
                o + rsrecv[last - 1, r0:r1, :].astype(F32)
            ).astype(BF)
            rs2[c].start()
        for c in range(NCH):
            r0, r1 = c * CH, (c + 1) * CH
            rs2[c].wait_recv()
            out_ref[r0:r1, :] = (
                out_ref[r0:r1, :] + rsrecv[last, r0:r1, :].astype(F32)
            )

        for r in ag + rs[:last] + rs2:
            r.wait_send()

    out = pl.pallas_call(
        body,
        out_shape=jax.ShapeDtypeStruct((SQ, D), F32),
        in_specs=[
            pl.BlockSpec(memory_space=pltpu.VMEM),
            pl.BlockSpec(memory_space=pltpu.VMEM),
            pl.BlockSpec(memory_space=pltpu.VMEM),
            pl.BlockSpec(memory_space=pltpu.MemorySpace.HBM),
            pl.BlockSpec(memory_space=pltpu.MemorySpace.HBM),
        ],
        out_specs=pl.BlockSpec(memory_space=pltpu.VMEM),
        scratch_shapes=[
            pltpu.VMEM((SQ, D), BF),
            pltpu.VMEM((N_DEV - 1, SQ, D), BF),
            pltpu.VMEM((N_DEV - 1, SQ, D), BF),
            pltpu.VMEM((N_DEV - 1, SQ, D), BF),
            pltpu.VMEM((D, D), BF),
            pltpu.VMEM((D, D), BF),
            pltpu.VMEM((SQ, D), BF),
            pltpu.VMEM((SQ, D), BF),
            pltpu.VMEM((2, SKV, DH), F32),
            pltpu.VMEM((2, SKV, DH), F32),
            pltpu.VMEM((H, SKV, DH), BF),
            pltpu.VMEM((H, SKV, DH), BF),
            pltpu.SemaphoreType.DMA((N_DEV - 1,)),
            pltpu.SemaphoreType.DMA((N_DEV - 1,)),
            pltpu.SemaphoreType.DMA((N_DEV - 1,)),
            pltpu.SemaphoreType.DMA((N_DEV - 1,)),
            pltpu.SemaphoreType.DMA((NCH,)),
            pltpu.SemaphoreType.DMA((NCH,)),
            pltpu.SemaphoreType.DMA((2,)),
            pltpu.SemaphoreType.DMA((2,)),
        ],
        compiler_params=pltpu.CompilerParams(
            collective_id=0, vmem_limit_bytes=63 * 1024 * 1024),
    )(x[0], Wq, Wo, K_ext[0], V_ext[0])
    return out[None]
